# Initial kernel scaffold; baseline (speedup 1.0000x reference)
import jax
import jax.numpy as jnp
from jax.experimental import pallas as pl


def kernel(x, one_hot, codebook_shared, codebook_task):
    raise NotImplementedError("write your pallas kernel here")


# speedup vs baseline: 1.7122x; 1.7122x over previous
import jax
import jax.numpy as jnp
from jax.experimental import pallas as pl
from jax.experimental.pallas import tpu as pltpu

_N_E_S = 512
_N_E_T = 128
_DS = 1024
_DT = 4
_TN = 512


def _vq_body(oh_ref, x_ref, cbs_ref, cbt_ref, out_ref, idxs_ref, idxt_ref, loss_ref):
    b = pl.program_id(0)
    zs = x_ref[0, 0:_DS, :]
    zt = x_ref[0, _DS:_DS + _DT, :]
    cb = cbs_ref[...]

    m = jax.lax.dot_general(cb, zs, (((1,), (0,)), ((), ())))
    zsq = jnp.sum(zs * zs, axis=0, keepdims=True)
    csq = jnp.sum(cb * cb, axis=1, keepdims=True)
    d = (zsq - 2.0 * m) + csq
    minv = jnp.min(d, axis=0, keepdims=True)
    iota = jax.lax.broadcasted_iota(jnp.int32, (_N_E_S, _TN), 0)
    idx = jnp.min(jnp.where(d == minv, iota, _N_E_S), axis=0)
    onehot = (iota == idx[None, :]).astype(jnp.float32)
    zq_s = jax.lax.dot_general(cb, onehot, (((0,), (0,)), ((), ())),
                               precision=jax.lax.Precision.HIGHEST)

    ct = (oh_ref[b, 0] * cbt_ref[0] + oh_ref[b, 1] * cbt_ref[1]
          + oh_ref[b, 2] * cbt_ref[2] + oh_ref[b, 3] * cbt_ref[3])
    mt = jax.lax.dot_general(ct, zt, (((1,), (0,)), ((), ())))
    ztsq = jnp.sum(zt * zt, axis=0, keepdims=True)
    ctsq = jnp.sum(ct * ct, axis=1, keepdims=True)
    dt = (ztsq - 2.0 * mt) + ctsq
    minvt = jnp.min(dt, axis=0, keepdims=True)
    iota_t = jax.lax.broadcasted_iota(jnp.int32, (_N_E_T, _TN), 0)
    idxt = jnp.min(jnp.where(dt == minvt, iota_t, _N_E_T), axis=0)
    onehot_t = (iota_t == idxt[None, :]).astype(jnp.float32)
    zq_t = jax.lax.dot_general(ct, onehot_t, (((0,), (0,)), ((), ())),
                               precision=jax.lax.Precision.HIGHEST)

    zq = jnp.concatenate([zq_s, zq_t], axis=0)
    xfull = x_ref[0]
    out_ref[0] = xfull + (zq - xfull)
    idxs_ref[0, 0, :] = idx
    idxt_ref[0, 0, :] = idxt
    loss_ref[...] = (jnp.sum(minv) + jnp.sum(minvt)).reshape(1, 1, 1, 1)


def kernel(x, one_hot, codebook_shared, codebook_task):
    B, C, D, H, W = x.shape
    N = D * H * W
    nt = N // _TN
    xr = x.reshape(B, C, N)

    grid = (B, nt)
    out_shape = [
        jax.ShapeDtypeStruct((B, C, N), jnp.float32),
        jax.ShapeDtypeStruct((B, 1, N), jnp.int32),
        jax.ShapeDtypeStruct((B, 1, N), jnp.int32),
        jax.ShapeDtypeStruct((B, nt, 1, 1), jnp.float32),
    ]
    in_specs = [
        pl.BlockSpec(memory_space=pltpu.SMEM),
        pl.BlockSpec((1, C, _TN), lambda b, t: (b, 0, t)),
        pl.BlockSpec((_N_E_S, _DS), lambda b, t: (0, 0)),
        pl.BlockSpec((4, _N_E_T, _DT), lambda b, t: (0, 0, 0)),
    ]
    out_specs = [
        pl.BlockSpec((1, C, _TN), lambda b, t: (b, 0, t)),
        pl.BlockSpec((1, 1, _TN), lambda b, t: (b, 0, t)),
        pl.BlockSpec((1, 1, _TN), lambda b, t: (b, 0, t)),
        pl.BlockSpec((1, 1, 1, 1), lambda b, t: (b, t, 0, 0)),
    ]
    out, idxs, idxt, lossp = pl.pallas_call(
        _vq_body,
        grid=grid,
        in_specs=in_specs,
        out_specs=out_specs,
        out_shape=out_shape,
        compiler_params=pltpu.CompilerParams(
            dimension_semantics=("parallel", "parallel"),
        ),
    )(one_hot, xr, codebook_shared, codebook_task)

    zq_fold = out.reshape(B, C, D, H, W)
    codebook_loss = 1.25 * jnp.sum(lossp) / (B * N * C)
    return zq_fold, codebook_loss, idxs.reshape(B, N), idxt.reshape(B, N)


# speedup vs baseline: 2.3794x; 1.3897x over previous
import jax
import jax.numpy as jnp
from jax.experimental import pallas as pl
from jax.experimental.pallas import tpu as pltpu

_N_E_S = 512
_N_E_T = 128
_DS = 1024
_DT = 4
_TN = 512


def _vq_body(oh_ref, x_ref, cbs_ref, cbt_ref, out_ref, idxs_ref, idxt_ref, loss_ref):
    b = pl.program_id(0)
    zs = x_ref[0, 0:_DS, :]
    zt = x_ref[0, _DS:_DS + _DT, :]
    cb = cbs_ref[...]

    m = jax.lax.dot_general(cb, zs, (((1,), (0,)), ((), ())))
    zsq = jnp.sum(zs * zs, axis=0, keepdims=True)
    csq = jnp.sum(cb * cb, axis=1, keepdims=True)
    d = (zsq - 2.0 * m) + csq
    minv = jnp.min(d, axis=0, keepdims=True)
    iota = jax.lax.broadcasted_iota(jnp.int32, (_N_E_S, _TN), 0)
    idx = jnp.min(jnp.where(d == minv, iota, _N_E_S), axis=0)
    onehot = (iota == idx[None, :]).astype(jnp.float32)
    zq_s = jax.lax.dot_general(cb, onehot, (((0,), (0,)), ((), ())))

    ct = (oh_ref[b, 0] * cbt_ref[0] + oh_ref[b, 1] * cbt_ref[1]
          + oh_ref[b, 2] * cbt_ref[2] + oh_ref[b, 3] * cbt_ref[3])
    mt = jax.lax.dot_general(ct, zt, (((1,), (0,)), ((), ())))
    ztsq = jnp.sum(zt * zt, axis=0, keepdims=True)
    ctsq = jnp.sum(ct * ct, axis=1, keepdims=True)
    dt = (ztsq - 2.0 * mt) + ctsq
    minvt = jnp.min(dt, axis=0, keepdims=True)
    iota_t = jax.lax.broadcasted_iota(jnp.int32, (_N_E_T, _TN), 0)
    idxt = jnp.min(jnp.where(dt == minvt, iota_t, _N_E_T), axis=0)
    onehot_t = (iota_t == idxt[None, :]).astype(jnp.float32)
    zq_t = jax.lax.dot_general(ct, onehot_t, (((0,), (0,)), ((), ())))

    zq = jnp.concatenate([zq_s, zq_t], axis=0)
    xfull = x_ref[0]
    out_ref[0] = xfull + (zq - xfull)
    idxs_ref[0, 0, :] = idx
    idxt_ref[0, 0, :] = idxt
    loss_ref[...] = (jnp.sum(minv) + jnp.sum(minvt)).reshape(1, 1, 1, 1)


def kernel(x, one_hot, codebook_shared, codebook_task):
    B, C, D, H, W = x.shape
    N = D * H * W
    nt = N // _TN
    xr = x.reshape(B, C, N)

    grid = (B, nt)
    out_shape = [
        jax.ShapeDtypeStruct((B, C, N), jnp.float32),
        jax.ShapeDtypeStruct((B, 1, N), jnp.int32),
        jax.ShapeDtypeStruct((B, 1, N), jnp.int32),
        jax.ShapeDtypeStruct((B, nt, 1, 1), jnp.float32),
    ]
    in_specs = [
        pl.BlockSpec(memory_space=pltpu.SMEM),
        pl.BlockSpec((1, C, _TN), lambda b, t: (b, 0, t)),
        pl.BlockSpec((_N_E_S, _DS), lambda b, t: (0, 0)),
        pl.BlockSpec((4, _N_E_T, _DT), lambda b, t: (0, 0, 0)),
    ]
    out_specs = [
        pl.BlockSpec((1, C, _TN), lambda b, t: (b, 0, t)),
        pl.BlockSpec((1, 1, _TN), lambda b, t: (b, 0, t)),
        pl.BlockSpec((1, 1, _TN), lambda b, t: (b, 0, t)),
        pl.BlockSpec((1, 1, 1, 1), lambda b, t: (b, t, 0, 0)),
    ]
    out, idxs, idxt, lossp = pl.pallas_call(
        _vq_body,
        grid=grid,
        in_specs=in_specs,
        out_specs=out_specs,
        out_shape=out_shape,
        compiler_params=pltpu.CompilerParams(
            dimension_semantics=("parallel", "parallel"),
        ),
    )(one_hot, xr, codebook_shared, codebook_task)

    zq_fold = out.reshape(B, C, D, H, W)
    codebook_loss = 1.25 * jnp.sum(lossp) / (B * N * C)
    return zq_fold, codebook_loss, idxs.reshape(B, N), idxt.reshape(B, N)


# speedup vs baseline: 2.5513x; 1.0722x over previous
import jax
import jax.numpy as jnp
from jax.experimental import pallas as pl
from jax.experimental.pallas import tpu as pltpu

_N_E_S = 512
_N_E_T = 128
_DS = 1024
_DT = 4
_TN = 1024


def _vq_body(oh_ref, x_ref, cbs_ref, cbt_ref, out_ref, idxs_ref, idxt_ref, loss_ref):
    b = pl.program_id(0)
    zs = x_ref[0, 0:_DS, :]
    zt = x_ref[0, _DS:_DS + _DT, :]
    cb = cbs_ref[...]

    m = jax.lax.dot_general(cb, zs, (((1,), (0,)), ((), ())))
    zsq = jnp.sum(zs * zs, axis=0, keepdims=True)
    csq = jnp.sum(cb * cb, axis=1, keepdims=True)
    d = (zsq - 2.0 * m) + csq
    minv = jnp.min(d, axis=0, keepdims=True)
    iota = jax.lax.broadcasted_iota(jnp.int32, (_N_E_S, _TN), 0)
    idx = jnp.min(jnp.where(d == minv, iota, _N_E_S), axis=0)
    onehot = (iota == idx[None, :]).astype(jnp.float32)
    zq_s = jax.lax.dot_general(cb, onehot, (((0,), (0,)), ((), ())))

    ct = (oh_ref[b, 0] * cbt_ref[0] + oh_ref[b, 1] * cbt_ref[1]
          + oh_ref[b, 2] * cbt_ref[2] + oh_ref[b, 3] * cbt_ref[3])
    mt = jax.lax.dot_general(ct, zt, (((1,), (0,)), ((), ())))
    ztsq = jnp.sum(zt * zt, axis=0, keepdims=True)
    ctsq = jnp.sum(ct * ct, axis=1, keepdims=True)
    dt = (ztsq - 2.0 * mt) + ctsq
    minvt = jnp.min(dt, axis=0, keepdims=True)
    iota_t = jax.lax.broadcasted_iota(jnp.int32, (_N_E_T, _TN), 0)
    idxt = jnp.min(jnp.where(dt == minvt, iota_t, _N_E_T), axis=0)
    onehot_t = (iota_t == idxt[None, :]).astype(jnp.float32)
    zq_t = jax.lax.dot_general(ct, onehot_t, (((0,), (0,)), ((), ())))

    out_ref[0, 0:_DS, :] = zs + (zq_s - zs)
    out_ref[0, _DS:_DS + _DT, :] = zt + (zq_t - zt)
    idxs_ref[0, 0, :] = idx
    idxt_ref[0, 0, :] = idxt
    loss_ref[...] = (jnp.sum(minv) + jnp.sum(minvt)).reshape(1, 1, 1, 1)


def kernel(x, one_hot, codebook_shared, codebook_task):
    B, C, D, H, W = x.shape
    N = D * H * W
    nt = N // _TN
    xr = x.reshape(B, C, N)

    grid = (B, nt)
    out_shape = [
        jax.ShapeDtypeStruct((B, C, N), jnp.float32),
        jax.ShapeDtypeStruct((B, 1, N), jnp.int32),
        jax.ShapeDtypeStruct((B, 1, N), jnp.int32),
        jax.ShapeDtypeStruct((B, nt, 1, 1), jnp.float32),
    ]
    in_specs = [
        pl.BlockSpec(memory_space=pltpu.SMEM),
        pl.BlockSpec((1, C, _TN), lambda b, t: (b, 0, t)),
        pl.BlockSpec((_N_E_S, _DS), lambda b, t: (0, 0)),
        pl.BlockSpec((4, _N_E_T, _DT), lambda b, t: (0, 0, 0)),
    ]
    out_specs = [
        pl.BlockSpec((1, C, _TN), lambda b, t: (b, 0, t)),
        pl.BlockSpec((1, 1, _TN), lambda b, t: (b, 0, t)),
        pl.BlockSpec((1, 1, _TN), lambda b, t: (b, 0, t)),
        pl.BlockSpec((1, 1, 1, 1), lambda b, t: (b, t, 0, 0)),
    ]
    out, idxs, idxt, lossp = pl.pallas_call(
        _vq_body,
        grid=grid,
        in_specs=in_specs,
        out_specs=out_specs,
        out_shape=out_shape,
        compiler_params=pltpu.CompilerParams(
            dimension_semantics=("parallel", "parallel"),
        ),
    )(one_hot, xr, codebook_shared, codebook_task)

    zq_fold = out.reshape(B, C, D, H, W)
    codebook_loss = 1.25 * jnp.sum(lossp) / (B * N * C)
    return zq_fold, codebook_loss, idxs.reshape(B, N), idxt.reshape(B, N)


# speedup vs baseline: 2.5955x; 1.0173x over previous
import jax
import jax.numpy as jnp
from jax.experimental import pallas as pl
from jax.experimental.pallas import tpu as pltpu

_N_E_S = 512
_N_E_T = 128
_DS = 1024
_DT = 4
_TN = 1536


def _vq_body(oh_ref, x_ref, cbs_ref, cbt_ref, out_ref, idxs_ref, idxt_ref, loss_ref):
    b = pl.program_id(0)
    zs = x_ref[0, 0:_DS, :]
    zt = x_ref[0, _DS:_DS + _DT, :]
    cb = cbs_ref[...]

    m = jax.lax.dot_general(cb, zs, (((1,), (0,)), ((), ())))
    zsq = jnp.sum(zs * zs, axis=0, keepdims=True)
    csq = jnp.sum(cb * cb, axis=1, keepdims=True)
    d = (zsq - 2.0 * m) + csq
    minv = jnp.min(d, axis=0, keepdims=True)
    iota = jax.lax.broadcasted_iota(jnp.int32, (_N_E_S, _TN), 0)
    idx = jnp.min(jnp.where(d == minv, iota, _N_E_S), axis=0)
    onehot = (iota == idx[None, :]).astype(jnp.float32)
    zq_s = jax.lax.dot_general(cb, onehot, (((0,), (0,)), ((), ())))

    ct = (oh_ref[b, 0] * cbt_ref[0] + oh_ref[b, 1] * cbt_ref[1]
          + oh_ref[b, 2] * cbt_ref[2] + oh_ref[b, 3] * cbt_ref[3])
    mt = jax.lax.dot_general(ct, zt, (((1,), (0,)), ((), ())))
    ztsq = jnp.sum(zt * zt, axis=0, keepdims=True)
    ctsq = jnp.sum(ct * ct, axis=1, keepdims=True)
    dt = (ztsq - 2.0 * mt) + ctsq
    minvt = jnp.min(dt, axis=0, keepdims=True)
    iota_t = jax.lax.broadcasted_iota(jnp.int32, (_N_E_T, _TN), 0)
    idxt = jnp.min(jnp.where(dt == minvt, iota_t, _N_E_T), axis=0)
    onehot_t = (iota_t == idxt[None, :]).astype(jnp.float32)
    zq_t = jax.lax.dot_general(ct, onehot_t, (((0,), (0,)), ((), ())))

    out_ref[0, 0:_DS, :] = zs + (zq_s - zs)
    out_ref[0, _DS:_DS + _DT, :] = zt + (zq_t - zt)
    idxs_ref[0, 0, :] = idx
    idxt_ref[0, 0, :] = idxt
    loss_ref[...] = (jnp.sum(minv) + jnp.sum(minvt)).reshape(1, 1, 1, 1)


def kernel(x, one_hot, codebook_shared, codebook_task):
    B, C, D, H, W = x.shape
    N = D * H * W
    nt = N // _TN
    xr = x.reshape(B, C, N)

    grid = (B, nt)
    out_shape = [
        jax.ShapeDtypeStruct((B, C, N), jnp.float32),
        jax.ShapeDtypeStruct((B, 1, N), jnp.int32),
        jax.ShapeDtypeStruct((B, 1, N), jnp.int32),
        jax.ShapeDtypeStruct((B, nt, 1, 1), jnp.float32),
    ]
    in_specs = [
        pl.BlockSpec(memory_space=pltpu.SMEM),
        pl.BlockSpec((1, C, _TN), lambda b, t: (b, 0, t)),
        pl.BlockSpec((_N_E_S, _DS), lambda b, t: (0, 0)),
        pl.BlockSpec((4, _N_E_T, _DT), lambda b, t: (0, 0, 0)),
    ]
    out_specs = [
        pl.BlockSpec((1, C, _TN), lambda b, t: (b, 0, t)),
        pl.BlockSpec((1, 1, _TN), lambda b, t: (b, 0, t)),
        pl.BlockSpec((1, 1, _TN), lambda b, t: (b, 0, t)),
        pl.BlockSpec((1, 1, 1, 1), lambda b, t: (b, t, 0, 0)),
    ]
    out, idxs, idxt, lossp = pl.pallas_call(
        _vq_body,
        grid=grid,
        in_specs=in_specs,
        out_specs=out_specs,
        out_shape=out_shape,
        compiler_params=pltpu.CompilerParams(
            dimension_semantics=("parallel", "parallel"),
        ),
    )(one_hot, xr, codebook_shared, codebook_task)

    zq_fold = out.reshape(B, C, D, H, W)
    codebook_loss = 1.25 * jnp.sum(lossp) / (B * N * C)
    return zq_fold, codebook_loss, idxs.reshape(B, N), idxt.reshape(B, N)


# speedup vs baseline: 2.6073x; 1.0045x over previous
import jax
import jax.numpy as jnp
from jax.experimental import pallas as pl
from jax.experimental.pallas import tpu as pltpu

_N_E_S = 512
_N_E_T = 128
_DS = 1024
_DT = 4
_TN = 1536


def _vq_body(oh_ref, x_ref, cbs_ref, cbt_ref, out_ref, idxs_ref, idxt_ref, loss_ref):
    b = pl.program_id(0)
    zs = x_ref[0, 0:_DS, :]
    zt = x_ref[0, _DS:_DS + _DT, :]
    cb = cbs_ref[...]

    m = jax.lax.dot_general(cb, zs, (((1,), (0,)), ((), ())))
    zsq = jnp.sum(zs * zs, axis=0, keepdims=True)
    csq = jnp.sum(cb * cb, axis=1, keepdims=True)
    d = (zsq - 2.0 * m) + csq
    minv = jnp.min(d, axis=0, keepdims=True)
    iota = jax.lax.broadcasted_iota(jnp.int32, (_N_E_S, _TN), 0)
    idx = jnp.min(jnp.where(d == minv, iota, _N_E_S), axis=0)
    onehot = (iota == idx[None, :]).astype(jnp.float32)
    zq_s = jax.lax.dot_general(cb, onehot, (((0,), (0,)), ((), ())))

    ct = (oh_ref[b, 0] * cbt_ref[0] + oh_ref[b, 1] * cbt_ref[1]
          + oh_ref[b, 2] * cbt_ref[2] + oh_ref[b, 3] * cbt_ref[3])
    mt = jax.lax.dot_general(ct, zt, (((1,), (0,)), ((), ())))
    ztsq = jnp.sum(zt * zt, axis=0, keepdims=True)
    ctsq = jnp.sum(ct * ct, axis=1, keepdims=True)
    dt = (ztsq - 2.0 * mt) + ctsq
    minvt = jnp.min(dt, axis=0, keepdims=True)
    iota_t = jax.lax.broadcasted_iota(jnp.int32, (_N_E_T, _TN), 0)
    idxt = jnp.min(jnp.where(dt == minvt, iota_t, _N_E_T), axis=0)
    onehot_t = (iota_t == idxt[None, :]).astype(jnp.float32)
    zq_t = jax.lax.dot_general(ct, onehot_t, (((0,), (0,)), ((), ())))

    out_ref[0, 0:_DS, :] = zq_s
    out_ref[0, _DS:_DS + _DT, :] = zq_t
    idxs_ref[0, 0, :] = idx
    idxt_ref[0, 0, :] = idxt
    loss_ref[...] = (jnp.sum(minv) + jnp.sum(minvt)).reshape(1, 1, 1, 1)


def kernel(x, one_hot, codebook_shared, codebook_task):
    B, C, D, H, W = x.shape
    N = D * H * W
    nt = N // _TN
    xr = x.reshape(B, C, N)

    grid = (B, nt)
    out_shape = [
        jax.ShapeDtypeStruct((B, C, N), jnp.float32),
        jax.ShapeDtypeStruct((B, 1, N), jnp.int32),
        jax.ShapeDtypeStruct((B, 1, N), jnp.int32),
        jax.ShapeDtypeStruct((B, nt, 1, 1), jnp.float32),
    ]
    in_specs = [
        pl.BlockSpec(memory_space=pltpu.SMEM),
        pl.BlockSpec((1, C, _TN), lambda b, t: (b, 0, t)),
        pl.BlockSpec((_N_E_S, _DS), lambda b, t: (0, 0)),
        pl.BlockSpec((4, _N_E_T, _DT), lambda b, t: (0, 0, 0)),
    ]
    out_specs = [
        pl.BlockSpec((1, C, _TN), lambda b, t: (b, 0, t)),
        pl.BlockSpec((1, 1, _TN), lambda b, t: (b, 0, t)),
        pl.BlockSpec((1, 1, _TN), lambda b, t: (b, 0, t)),
        pl.BlockSpec((1, 1, 1, 1), lambda b, t: (b, t, 0, 0)),
    ]
    out, idxs, idxt, lossp = pl.pallas_call(
        _vq_body,
        grid=grid,
        in_specs=in_specs,
        out_specs=out_specs,
        out_shape=out_shape,
        compiler_params=pltpu.CompilerParams(
            dimension_semantics=("parallel", "parallel"),
        ),
    )(one_hot, xr, codebook_shared, codebook_task)

    zq_fold = out.reshape(B, C, D, H, W)
    codebook_loss = 1.25 * jnp.sum(lossp) / (B * N * C)
    return zq_fold, codebook_loss, idxs.reshape(B, N), idxt.reshape(B, N)
